# baseline (device time: 21369 ns/iter reference)
import functools

import jax
import jax.numpy as jnp
from jax import lax
from jax.experimental import pallas as pl
from jax.experimental.pallas import tpu as pltpu

N_DEV = 4
BLK = 64


def kernel(x, Wq, K_ext, V_ext, Wo):
    B, Sq, Dm = x.shape
    _, Skv, Hq, Dh = K_ext.shape
    HD = Hq * Dh
    Dout = Wo.shape[1]
    J = Sq // BLK

    K2 = K_ext.reshape(B, Skv, HD)
    V2 = V_ext.reshape(B, Skv, HD)

    def body(x_ref, wq_ref, k_ref, v_ref, wo_ref, out_ref,
             ksel, vsel, ksend, krecv, vsend, vrecv):
        my = lax.axis_index("i")
        peers = [(my + d) % N_DEV for d in (1, 2, 3)]

        barrier = pltpu.get_barrier_semaphore()
        for p in peers:
            pl.semaphore_signal(barrier, inc=1, device_id=(p,),
                                device_id_type=pl.DeviceIdType.MESH)
        pl.semaphore_wait(barrier, 3)

        ksel[:, :, 0] = k_ref[...].astype(jnp.bfloat16).reshape(B, J, BLK, HD)
        vsel[:, :, 0] = v_ref[...].astype(jnp.bfloat16).reshape(B, J, BLK, HD)

        rdmas = []
        for d in (1, 2, 3):
            t = N_DEV - d
            rk = pltpu.make_async_remote_copy(
                src_ref=ksel.at[:, :, 0], dst_ref=ksel.at[:, :, t],
                send_sem=ksend.at[d], recv_sem=krecv.at[t],
                device_id=(peers[d - 1],),
                device_id_type=pl.DeviceIdType.MESH)
            rv = pltpu.make_async_remote_copy(
                src_ref=vsel.at[:, :, 0], dst_ref=vsel.at[:, :, t],
                send_sem=vsend.at[d], recv_sem=vrecv.at[t],
                device_id=(peers[d - 1],),
                device_id_type=pl.DeviceIdType.MESH)
            rk.start()
            rv.start()
            rdmas.extend((rk, rv))

        wq = wq_ref[...].astype(jnp.bfloat16)
        wo = wo_ref[...].astype(jnp.bfloat16)
        q = [(jnp.dot(x_ref[b].astype(jnp.bfloat16), wq,
                      preferred_element_type=jnp.float32) * 0.125
              ).astype(jnp.bfloat16) for b in range(B)]

        for r in rdmas:
            r.wait_recv()

        for b in range(B):
            ctx_rows = []
            for j in range(J):
                k_sel = ksel[b, j].reshape(N_DEV * BLK, HD)
                v_sel = vsel[b, j].reshape(N_DEV * BLK, HD)
                q_blk = q[b][j * BLK:(j + 1) * BLK, :]
                ctx_heads = []
                for hh in range(Hq):
                    cs = slice(hh * Dh, (hh + 1) * Dh)
                    s = lax.dot_general(
                        q_blk[:, cs], k_sel[:, cs],
                        (((1,), (1,)), ((), ())),
                        preferred_element_type=jnp.float32)
                    e = jnp.exp(s)
                    l = jnp.sum(e, axis=-1, keepdims=True)
                    ctx = jnp.dot(e.astype(jnp.bfloat16), v_sel[:, cs],
                                  preferred_element_type=jnp.float32)
                    ctx_heads.append((ctx * (1.0 / l)).astype(jnp.bfloat16))
                ctx_rows.append(jnp.concatenate(ctx_heads, axis=1))
            ctx_b = jnp.concatenate(ctx_rows, axis=0)
            out_ref[b] = jnp.dot(ctx_b, wo, preferred_element_type=jnp.float32)

        for r in rdmas:
            r.wait_send()

        @functools.partial(pl.run_scoped,
                           second_barrier=pltpu.SemaphoreType.REGULAR)
        def _(second_barrier):
            for p in peers:
                pl.semaphore_signal(second_barrier, inc=1, device_id=(p,),
                                    device_id_type=pl.DeviceIdType.MESH)
            pl.semaphore_wait(second_barrier, 3)

    return pl.pallas_call(
        body,
        out_shape=jax.ShapeDtypeStruct((B, Sq, Dout), jnp.float32),
        in_specs=[pl.BlockSpec(memory_space=pltpu.VMEM)] * 5,
        out_specs=pl.BlockSpec(memory_space=pltpu.VMEM),
        scratch_shapes=[
            pltpu.VMEM((B, J, N_DEV, BLK, HD), jnp.bfloat16),
            pltpu.VMEM((B, J, N_DEV, BLK, HD), jnp.bfloat16),
            pltpu.SemaphoreType.DMA((N_DEV,)),
            pltpu.SemaphoreType.DMA((N_DEV,)),
            pltpu.SemaphoreType.DMA((N_DEV,)),
            pltpu.SemaphoreType.DMA((N_DEV,)),
        ],
        compiler_params=pltpu.CompilerParams(collective_id=0),
    )(x, Wq, K2, V2, Wo)
